# baseline (speedup 1.0000x reference)
import functools
import jax
import jax.numpy as jnp
from jax import lax
from jax.experimental import pallas as pl
from jax.experimental.pallas import tpu as pltpu
from jax.experimental.pallas import tpu_sc as plsc

IMAGE_H = 400
IMAGE_W = 400
N_RAYS = 1024
N_PTS = 64
MIN_DEPTH = 0.1
MAX_DEPTH = 10.0
NBLK = 25


def _prep_body(mask_hbm, rowcdf_ref, bext_ref, x_vmem, dma_sem):
    b = pl.program_id(0)
    cp = pltpu.make_async_copy(mask_hbm.at[b, :, pl.ds(0, 896)], x_vmem, dma_sem)
    cp.start()
    cp.wait()
    x = x_vmem[...]
    lane = lax.broadcasted_iota(jnp.int32, (IMAGE_H, 896), 1)
    xm = jnp.where(((lane % 2) == 0) & (lane < 800), x, 0.0)

    s_i = lax.broadcasted_iota(jnp.int32, (896, NBLK), 0)
    s_j = lax.broadcasted_iota(jnp.int32, (896, NBLK), 1)
    sel = ((s_i // 32) == s_j).astype(jnp.float32)
    bsum = lax.dot_general(
        xm, sel, (((1,), (0,)), ((), ())),
        preferred_element_type=jnp.float32, precision=lax.Precision.HIGHEST,
    )

    u_i = lax.broadcasted_iota(jnp.int32, (NBLK, NBLK), 0)
    u_j = lax.broadcasted_iota(jnp.int32, (NBLK, NBLK), 1)
    upper = (u_i <= u_j).astype(jnp.float32)
    bpre = lax.dot_general(
        bsum, upper, (((1,), (0,)), ((), ())),
        preferred_element_type=jnp.float32, precision=lax.Precision.HIGHEST,
    )
    rowsum = bpre[:, NBLK - 1 : NBLK]

    r_i = lax.broadcasted_iota(jnp.int32, (IMAGE_H, IMAGE_H), 0)
    r_j = lax.broadcasted_iota(jnp.int32, (IMAGE_H, IMAGE_H), 1)
    l_incl = (r_i <= r_j).astype(jnp.float32)
    l_excl = (r_i < r_j).astype(jnp.float32)
    rowcdf_row = lax.dot_general(
        rowsum, l_incl, (((0,), (0,)), ((), ())),
        preferred_element_type=jnp.float32, precision=lax.Precision.HIGHEST,
    )
    rowprev_row = lax.dot_general(
        rowsum, l_excl, (((0,), (0,)), ((), ())),
        preferred_element_type=jnp.float32, precision=lax.Precision.HIGHEST,
    )
    rowprev_col = jnp.transpose(rowprev_row)

    rowcdf_ref[0] = rowcdf_row
    bext_ref[0] = jnp.concatenate([rowprev_col, bpre + rowprev_col], axis=1)


def _tc_prep(mr, B):
    return pl.pallas_call(
        _prep_body,
        grid=(B,),
        in_specs=[pl.BlockSpec(memory_space=pl.ANY)],
        out_specs=[
            pl.BlockSpec((1, 1, IMAGE_H), lambda b: (b, 0, 0)),
            pl.BlockSpec((1, IMAGE_H, 26), lambda b: (b, 0, 0)),
        ],
        out_shape=[
            jax.ShapeDtypeStruct((B, 1, IMAGE_H), jnp.float32),
            jax.ShapeDtypeStruct((B, IMAGE_H, 26), jnp.float32),
        ],
        scratch_shapes=[
            pltpu.VMEM((IMAGE_H, 896), jnp.float32),
            pltpu.SemaphoreType.DMA,
        ],
    )(mr)


def _sc_search(rowcdf1d, bext1d, table, u1d, B):
    mesh = plsc.VectorSubcoreMesh(core_axis_name="c", subcore_axis_name="s")
    RPW = 512

    @functools.partial(
        pl.kernel,
        mesh=mesh,
        out_type=jax.ShapeDtypeStruct((B * N_RAYS,), jnp.int32),
        compiler_params=pltpu.CompilerParams(
            use_tc_tiling_on_sc=False, needs_layout_passes=False
        ),
        scratch_types=[
            pltpu.VMEM((IMAGE_H,), jnp.float32),
            pltpu.VMEM((IMAGE_H * 26,), jnp.float32),
            pltpu.VMEM((RPW,), jnp.float32),
            pltpu.VMEM((4, 128), jnp.int32),
            pltpu.VMEM((4, 128, 32), jnp.float32),
            pltpu.VMEM((RPW,), jnp.int32),
            pltpu.VMEM((RPW,), jnp.int32),
            pltpu.VMEM((RPW,), jnp.float32),
            pltpu.VMEM((RPW,), jnp.float32),
            pltpu.VMEM((RPW,), jnp.int32),
            pltpu.SemaphoreType.DMA,
        ],
    )
    def k(rowcdf_hbm, bext_hbm, table_hbm, u_hbm, out_hbm,
          rowcdf_v, bext_v, u_v, cidx_v, chunks_v, rows_v, cbase_v,
          prev_v, vv_v, out_v, sem):
        wid = lax.axis_index("s") * 2 + lax.axis_index("c")
        cam = lax.shift_right_logical(wid, 1)
        pltpu.sync_copy(rowcdf_hbm.at[pl.ds(cam * IMAGE_H, IMAGE_H)], rowcdf_v)
        pltpu.sync_copy(bext_hbm.at[pl.ds(cam * (IMAGE_H * 26), IMAGE_H * 26)],
                        bext_v)
        pltpu.sync_copy(u_hbm.at[pl.ds(wid * RPW, RPW)], u_v)

        total_idx = jnp.full((16,), IMAGE_H - 1, jnp.int32)
        total = plsc.load_gather(rowcdf_v, [total_idx])
        cam_base = cam * 20000

        def search_body(i, _):
            off = pl.multiple_of(i * 16, 16)
            sl = pl.ds(off, 16)
            uu = u_v[sl]
            v = uu * total

            lo = jnp.zeros((16,), jnp.int32)
            for step in (256, 128, 64, 32, 16, 8, 4, 2, 1):
                probe = lo + (step - 1)
                pc = jnp.minimum(probe, IMAGE_H - 1)
                val = plsc.load_gather(rowcdf_v, [pc])
                cond = (probe < IMAGE_H) & (val < v)
                lo = lo + jnp.where(cond, step, 0)
            row = jnp.minimum(lo, IMAGE_H - 1)

            base = row * 26
            bo = jnp.zeros((16,), jnp.int32)
            for step in (16, 8, 4, 2, 1):
                probe = bo + (step - 1)
                pc = jnp.minimum(probe, NBLK - 1)
                val = plsc.load_gather(bext_v, [base + 1 + pc])
                cond = (probe < NBLK) & (val < v)
                bo = bo + jnp.where(cond, step, 0)
            blk = jnp.minimum(bo, NBLK - 1)
            prev = plsc.load_gather(bext_v, [base + blk])

            rows_v[sl] = row
            cbase_v[sl] = blk * 16
            prev_v[sl] = prev
            vv_v[sl] = v
            j = lax.shift_right_logical(i, 3)
            joff = pl.multiple_of((i & 7) * 16, 16)
            cidx_v[j, pl.ds(joff, 16)] = cam_base + row * 50 + blk
            return 0

        lax.fori_loop(0, RPW // 16, search_body, 0)

        copies = [
            pltpu.async_copy(table_hbm.at[cidx_v.at[j]], chunks_v.at[j], sem)
            for j in range(4)
        ]
        for c in copies:
            c.wait()

        lane = lax.iota(jnp.int32, 16)

        def count_body(i, _):
            off = pl.multiple_of(i * 16, 16)
            sl = pl.ds(off, 16)
            row = rows_v[sl]
            cbase = cbase_v[sl]
            prev = prev_v[sl]
            v = vv_v[sl]
            slot = lane + (i & 7) * 16
            jv = jnp.zeros((16,), jnp.int32) + lax.shift_right_logical(i, 3)
            acc = jnp.zeros((16,), jnp.float32)
            cnt = jnp.zeros((16,), jnp.int32)
            for kk in range(16):
                elem = jnp.full((16,), 2 * kk, jnp.int32)
                val = plsc.load_gather(chunks_v, [jv, slot, elem])
                acc = acc + val
                cnt = cnt + jnp.where((prev + acc) < v, 1, 0)
            col = jnp.minimum(cbase + cnt, IMAGE_W - 1)
            out_v[sl] = row * IMAGE_W + col
            return 0

        lax.fori_loop(0, RPW // 16, count_body, 0)

        pltpu.sync_copy(out_v, out_hbm.at[pl.ds(wid * RPW, RPW)])

    return k(rowcdf1d, bext1d, table, u1d)


def _fin_body(idx_ref, R_ref, p_ref, o_ref):
    idx = idx_ref[0]
    row = (idx // IMAGE_W).astype(jnp.float32)
    col = (idx % IMAGE_W).astype(jnp.float32)

    half_x = 1.0 / IMAGE_W
    half_y = 1.0 / IMAGE_H
    step_x = jnp.float32((-1.0 + half_x - (1.0 - half_x)) / (IMAGE_W - 1))
    step_y = jnp.float32((-1.0 + half_y - (1.0 - half_y)) / (IMAGE_H - 1))
    xx = jnp.float32(1.0 - half_x) + col * step_x
    yy = jnp.float32(1.0 - half_y) + row * step_y

    fx = p_ref[0, 0, 3]; fy = p_ref[0, 0, 4]
    ppx = p_ref[0, 0, 5]; ppy = p_ref[0, 0, 6]
    d0 = (xx - ppx) / fx
    d1 = (yy - ppy) / fy
    R00 = R_ref[0, 0, 0]; R01 = R_ref[0, 0, 1]; R02 = R_ref[0, 0, 2]
    R10 = R_ref[0, 1, 0]; R11 = R_ref[0, 1, 1]; R12 = R_ref[0, 1, 2]
    R20 = R_ref[0, 2, 0]; R21 = R_ref[0, 2, 1]; R22 = R_ref[0, 2, 2]
    w0 = d0 * R00 + d1 * R01 + R02
    w1 = d0 * R10 + d1 * R11 + R12
    w2 = d0 * R20 + d1 * R21 + R22
    inv = lax.rsqrt(w0 * w0 + w1 * w1 + w2 * w2)

    t0 = p_ref[0, 0, 0]; t1 = p_ref[0, 0, 1]; t2 = p_ref[0, 0, 2]
    o0 = -(t0 * R00 + t1 * R01 + t2 * R02)
    o1 = -(t0 * R10 + t1 * R11 + t2 * R12)
    o2 = -(t0 * R20 + t1 * R21 + t2 * R22)
    ones = jnp.ones((N_RAYS, 1), jnp.float32)

    d_iota = lax.broadcasted_iota(jnp.int32, (N_RAYS, N_PTS), 1).astype(jnp.float32)
    d_step = jnp.float32((MAX_DEPTH - MIN_DEPTH) / (N_PTS - 1))
    lengths = jnp.float32(MIN_DEPTH) + d_iota * d_step

    o_ref[0] = jnp.concatenate(
        [o0 * ones, o1 * ones, o2 * ones, w0 * inv, w1 * inv, w2 * inv,
         lengths, xx, yy],
        axis=1,
    )


def _tc_finalize(idx, R, params, B):
    return pl.pallas_call(
        _fin_body,
        grid=(B,),
        in_specs=[
            pl.BlockSpec((1, N_RAYS, 1), lambda b: (b, 0, 0)),
            pl.BlockSpec((1, 3, 3), lambda b: (b, 0, 0)),
            pl.BlockSpec((1, 1, 8), lambda b: (b, 0, 0)),
        ],
        out_specs=pl.BlockSpec((1, N_RAYS, 72), lambda b: (b, 0, 0)),
        out_shape=jax.ShapeDtypeStruct((B, N_RAYS, 72), jnp.float32),
    )(idx.reshape(B, N_RAYS, 1), R, params)


def kernel(mask, R, T, focal, principal_point):
    B = mask.shape[0]
    mr = mask[:, 0].reshape(B, IMAGE_H, 1600)
    u = jax.random.uniform(jax.random.key(42), (B, N_RAYS), dtype=jnp.float32)
    params = jnp.concatenate(
        [T, focal, principal_point, jnp.zeros((B, 1), jnp.float32)], axis=1
    ).reshape(B, 1, 8)

    rowcdf3, bext = _tc_prep(mr, B)
    rowcdf1d = rowcdf3.reshape(B * IMAGE_H)
    bext1d = bext.reshape(B * IMAGE_H * 26)
    table = mask.reshape(B * 20000, 32)
    idx = _sc_search(rowcdf1d, bext1d, table, u.reshape(B * N_RAYS), B)
    return _tc_finalize(idx.reshape(B, N_RAYS), R, params, B)
